# baseline (speedup 1.0000x reference)
import functools
import math

import jax
import jax.numpy as jnp
from jax import lax
from jax.experimental import pallas as pl
from jax.experimental.pallas import tpu as pltpu
from jax.experimental.pallas import tpu_sc as plsc

D = 2048
B_TOTAL = 4 * 8192
NC = 2
NS = 16
NW = NC * NS
ROWS_PER_W = B_TOTAL // NW
CHUNK = 16
NCHUNK = ROWS_PER_W // CHUNK
SCALE = math.sqrt(float(D))

_mesh = plsc.VectorSubcoreMesh(core_axis_name="c", subcore_axis_name="s")


@functools.partial(
    pl.kernel,
    mesh=_mesh,
    out_type=jax.ShapeDtypeStruct((B_TOTAL, D), jnp.float32),
    scratch_types=[
        pltpu.VMEM((NCHUNK, CHUNK), jnp.int32),
        pltpu.VMEM((CHUNK, D), jnp.float32),
        pltpu.VMEM((CHUNK, D), jnp.float32),
        pltpu.SemaphoreType.DMA,
        pltpu.SemaphoreType.DMA,
        pltpu.SemaphoreType.DMA,
        pltpu.SemaphoreType.DMA,
    ],
)
def _embed_sc(idx_hbm, tab_hbm, out_hbm, idx_v, buf0, buf1, g0, g1, o0, o1):
    wid = lax.axis_index("s") * NC + lax.axis_index("c")
    base = wid * ROWS_PER_W

    bufs = (buf0, buf1)
    gsems = (g0, g1)
    osems = (o0, o1)

    pltpu.sync_copy(idx_hbm.at[wid], idx_v)

    pltpu.async_copy(tab_hbm.at[idx_v[0]], buf0, g0)
    pltpu.async_copy(tab_hbm.at[idx_v[1]], buf1, g1)

    def step(i, carry):
        for b in range(2):
            j = i * 2 + b
            buf = bufs[b]
            pltpu.make_async_copy(tab_hbm.at[pl.ds(0, CHUNK)], buf, gsems[b]).wait()

            def srow(r, _):
                def scol(c, _):
                    sl = pl.ds(c * 16, 16)
                    buf[r, sl] = buf[r, sl] * SCALE
                    return 0
                return lax.fori_loop(0, D // 16, scol, 0)
            lax.fori_loop(0, CHUNK, srow, 0)

            pltpu.async_copy(buf, out_hbm.at[pl.ds(base + j * CHUNK, CHUNK)], osems[b])

            @pl.when(j < NCHUNK - 2)
            def _():
                pltpu.make_async_copy(
                    buf, out_hbm.at[pl.ds(base, CHUNK)], osems[b]).wait()
                pltpu.async_copy(tab_hbm.at[idx_v[j + 2]], buf, gsems[b])
        return carry

    lax.fori_loop(0, NCHUNK // 2, step, 0)

    pltpu.make_async_copy(buf0, out_hbm.at[pl.ds(base, CHUNK)], o0).wait()
    pltpu.make_async_copy(buf1, out_hbm.at[pl.ds(base, CHUNK)], o1).wait()


def kernel(sequence, table):
    seq = sequence.reshape(-1).astype(jnp.int32).reshape(NW, NCHUNK, CHUNK)
    out = _embed_sc(seq, table)
    return out.reshape(sequence.shape + (D,))


# speedup vs baseline: 3.1315x; 3.1315x over previous
import functools
import math

import jax
import jax.numpy as jnp
from jax import lax
from jax.experimental import pallas as pl
from jax.experimental.pallas import tpu as pltpu
from jax.experimental.pallas import tpu_sc as plsc

D = 2048
B_TOTAL = 4 * 8192
NC = 2
NS = 16
NW = NC * NS
ROWS_PER_W = B_TOTAL // NW
CHUNK = 16
NCHUNK = ROWS_PER_W // CHUNK
SCALE = math.sqrt(float(D))

_mesh = plsc.VectorSubcoreMesh(core_axis_name="c", subcore_axis_name="s")


@functools.partial(
    pl.kernel,
    mesh=_mesh,
    out_type=jax.ShapeDtypeStruct((B_TOTAL, D), jnp.float32),
    scratch_types=[
        pltpu.VMEM((NCHUNK, CHUNK), jnp.int32),
        pltpu.VMEM((CHUNK, D), jnp.float32),
        pltpu.VMEM((CHUNK, D), jnp.float32),
        pltpu.SemaphoreType.DMA,
        pltpu.SemaphoreType.DMA,
        pltpu.SemaphoreType.DMA,
        pltpu.SemaphoreType.DMA,
    ],
)
def _embed_sc(idx_hbm, tab_hbm, out_hbm, idx_v, buf0, buf1, g0, g1, o0, o1):
    wid = lax.axis_index("s") * NC + lax.axis_index("c")
    base = wid * ROWS_PER_W

    bufs = (buf0, buf1)
    gsems = (g0, g1)
    osems = (o0, o1)

    pltpu.sync_copy(idx_hbm.at[wid], idx_v)

    pltpu.async_copy(tab_hbm.at[idx_v[0]], buf0, g0)
    pltpu.async_copy(tab_hbm.at[idx_v[1]], buf1, g1)

    def step(i, carry):
        for b in range(2):
            j = i * 2 + b
            buf = bufs[b]
            pltpu.make_async_copy(tab_hbm.at[pl.ds(0, CHUNK)], buf, gsems[b]).wait()

            def srow(r, _):
                for c in range(D // 16):
                    sl = pl.ds(c * 16, 16)
                    buf[r, sl] = buf[r, sl] * SCALE
                return 0
            lax.fori_loop(0, CHUNK, srow, 0)

            pltpu.async_copy(buf, out_hbm.at[pl.ds(base + j * CHUNK, CHUNK)], osems[b])

            @pl.when(j < NCHUNK - 2)
            def _():
                pltpu.make_async_copy(
                    buf, out_hbm.at[pl.ds(base, CHUNK)], osems[b]).wait()
                pltpu.async_copy(tab_hbm.at[idx_v[j + 2]], buf, gsems[b])
        return carry

    lax.fori_loop(0, NCHUNK // 2, step, 0)

    pltpu.make_async_copy(buf0, out_hbm.at[pl.ds(base, CHUNK)], o0).wait()
    pltpu.make_async_copy(buf1, out_hbm.at[pl.ds(base, CHUNK)], o1).wait()


def kernel(sequence, table):
    seq = sequence.reshape(-1).astype(jnp.int32).reshape(NW, NCHUNK, CHUNK)
    out = _embed_sc(seq, table)
    return out.reshape(sequence.shape + (D,))


# speedup vs baseline: 3.6925x; 1.1792x over previous
import functools
import math

import jax
import jax.numpy as jnp
from jax import lax
from jax.experimental import pallas as pl
from jax.experimental.pallas import tpu as pltpu
from jax.experimental.pallas import tpu_sc as plsc

D = 2048
B_TOTAL = 4 * 8192
NC = 2
NS = 16
NW = NC * NS
ROWS_PER_W = B_TOTAL // NW
CHUNK = 8
NCHUNK = ROWS_PER_W // CHUNK
NBUF = 4
SCALE = math.sqrt(float(D))

_mesh = plsc.VectorSubcoreMesh(core_axis_name="c", subcore_axis_name="s")


@functools.partial(
    pl.kernel,
    mesh=_mesh,
    out_type=jax.ShapeDtypeStruct((B_TOTAL, D), jnp.float32),
    scratch_types=(
        [pltpu.VMEM((NCHUNK, CHUNK), jnp.int32)]
        + [pltpu.VMEM((CHUNK, D), jnp.float32) for _ in range(NBUF)]
        + [pltpu.SemaphoreType.DMA for _ in range(2 * NBUF)]
    ),
)
def _embed_sc(idx_hbm, tab_hbm, out_hbm, idx_v, b0, b1, b2, b3,
              g0, g1, g2, g3, o0, o1, o2, o3):
    wid = lax.axis_index("s") * NC + lax.axis_index("c")
    base = wid * ROWS_PER_W

    bufs = (b0, b1, b2, b3)
    gsems = (g0, g1, g2, g3)
    osems = (o0, o1, o2, o3)

    pltpu.sync_copy(idx_hbm.at[wid], idx_v)

    pltpu.async_copy(tab_hbm.at[idx_v.at[0]], b0, g0)
    pltpu.async_copy(tab_hbm.at[idx_v.at[1]], b1, g1)

    def step(i, carry):
        for b in range(NBUF):
            j = i * NBUF + b
            pb = (b + 2) % NBUF

            @pl.when(jnp.logical_and(j >= 2, j + 2 < NCHUNK))
            def _():
                pltpu.make_async_copy(
                    bufs[pb], out_hbm.at[pl.ds(base, CHUNK)], osems[pb]).wait()

            @pl.when(j + 2 < NCHUNK)
            def _():
                pltpu.async_copy(tab_hbm.at[idx_v.at[j + 2]], bufs[pb], gsems[pb])

            buf = bufs[b]
            pltpu.make_async_copy(
                tab_hbm.at[pl.ds(0, CHUNK)], buf, gsems[b]).wait()

            def srow(r, _):
                for c in range(D // 16):
                    sl = pl.ds(c * 16, 16)
                    buf[r, sl] = buf[r, sl] * SCALE
                return 0
            lax.fori_loop(0, CHUNK, srow, 0)

            pltpu.async_copy(
                buf, out_hbm.at[pl.ds(base + j * CHUNK, CHUNK)], osems[b])
        return carry

    lax.fori_loop(0, NCHUNK // NBUF, step, 0)

    for b in range(NBUF):
        pltpu.make_async_copy(
            bufs[b], out_hbm.at[pl.ds(base, CHUNK)], osems[b]).wait()


def kernel(sequence, table):
    seq = sequence.reshape(-1).astype(jnp.int32).reshape(NW, NCHUNK, CHUNK)
    out = _embed_sc(seq, table)
    return out.reshape(sequence.shape + (D,))
